# baseline (device time: 89707 ns/iter reference)
import jax
import jax.numpy as jnp
from jax import lax
from jax.experimental import pallas as pl
from jax.experimental.pallas import tpu as pltpu

N_DEV = 4


def kernel(A, B):
    m, k = A.shape
    k2, n = B.shape

    def body(a_ref, b_ref, out_ref, comm_ref, send_sems, recv_sems):
        my_pos = lax.axis_index("i")
        left = (my_pos - 1) % N_DEV
        right = (my_pos + 1) % N_DEV

        barrier_sem = pltpu.get_barrier_semaphore()
        for nbr in [left, right]:
            pl.semaphore_signal(
                barrier_sem, inc=1,
                device_id=(nbr,), device_id_type=pl.DeviceIdType.MESH,
            )
        pl.semaphore_wait(barrier_sem, 2)

        partial = jnp.dot(
            a_ref[:, :].astype(jnp.bfloat16),
            b_ref[:, :].astype(jnp.bfloat16),
            preferred_element_type=jnp.float32,
        )
        out_ref[:, :] = partial
        comm_ref[0, :, :] = partial

        for h in range(N_DEV - 1):
            send_slot = h % 2
            recv_slot = (h + 1) % 2
            rdma = pltpu.make_async_remote_copy(
                src_ref=comm_ref.at[send_slot],
                dst_ref=comm_ref.at[recv_slot],
                send_sem=send_sems.at[send_slot],
                recv_sem=recv_sems.at[recv_slot],
                device_id=(right,),
                device_id_type=pl.DeviceIdType.MESH,
            )
            rdma.start()
            rdma.wait()

            out_ref[:, :] += comm_ref[recv_slot, :, :]

    return pl.pallas_call(
        body,
        out_shape=jax.ShapeDtypeStruct((m, n), jnp.float32),
        in_specs=[
            pl.BlockSpec(memory_space=pltpu.VMEM),
            pl.BlockSpec(memory_space=pltpu.VMEM),
        ],
        out_specs=pl.BlockSpec(memory_space=pltpu.VMEM),
        scratch_shapes=[
            pltpu.VMEM((2, m, n), jnp.float32),
            pltpu.SemaphoreType.DMA((2,)),
            pltpu.SemaphoreType.DMA((2,)),
        ],
        compiler_params=pltpu.CompilerParams(collective_id=0),
    )(A, B)


# device time: 24841 ns/iter; 3.6112x vs baseline; 3.6112x over previous
import jax
import jax.numpy as jnp
from jax import lax
from jax.experimental import pallas as pl
from jax.experimental.pallas import tpu as pltpu

N_DEV = 4


def kernel(A, B):
    m, k = A.shape
    k2, n = B.shape
    q = m // N_DEV

    def body(a_ref, b_ref, out_ref, stage_ref, red_ref, comm1_ref, comm2_ref,
             send_sems1, recv_sems1, send_sems2, recv_sems2):
        my_pos = lax.axis_index("i")

        barrier_sem = pltpu.get_barrier_semaphore()
        for off in range(1, N_DEV):
            pl.semaphore_signal(
                barrier_sem, inc=1,
                device_id=((my_pos + off) % N_DEV,),
                device_id_type=pl.DeviceIdType.MESH,
            )
        pl.semaphore_wait(barrier_sem, N_DEV - 1)

        partial = jnp.dot(
            a_ref[:, :].astype(jnp.bfloat16),
            b_ref[:, :].astype(jnp.bfloat16),
            preferred_element_type=jnp.float32,
        )
        stage_ref[:, :] = partial.astype(jnp.bfloat16)
        out_ref[:, :] = partial

        p1 = []
        for off in range(1, N_DEV):
            d = (my_pos + off) % N_DEV
            rdma = pltpu.make_async_remote_copy(
                src_ref=stage_ref.at[pl.ds(d * q, q), :],
                dst_ref=comm1_ref.at[off - 1],
                send_sem=send_sems1.at[off - 1],
                recv_sem=recv_sems1.at[off - 1],
                device_id=(d,),
                device_id_type=pl.DeviceIdType.MESH,
            )
            rdma.start()
            p1.append(rdma)

        for rdma in p1:
            rdma.wait_recv()
        acc = out_ref[pl.ds(my_pos * q, q), :]
        for off in range(1, N_DEV):
            acc = acc + comm1_ref[off - 1, :, :].astype(jnp.float32)
        out_ref[pl.ds(my_pos * q, q), :] = acc
        red_ref[:, :] = acc.astype(jnp.bfloat16)

        p2 = []
        for off in range(1, N_DEV):
            d = (my_pos + off) % N_DEV
            rdma = pltpu.make_async_remote_copy(
                src_ref=red_ref,
                dst_ref=comm2_ref.at[off - 1],
                send_sem=send_sems2.at[off - 1],
                recv_sem=recv_sems2.at[off - 1],
                device_id=(d,),
                device_id_type=pl.DeviceIdType.MESH,
            )
            rdma.start()
            p2.append(rdma)

        for off in range(1, N_DEV):
            p2[off - 1].wait_recv()
            src = (my_pos - off) % N_DEV
            out_ref[pl.ds(src * q, q), :] = (
                comm2_ref[off - 1, :, :].astype(jnp.float32)
            )

        for rdma in p1:
            rdma.wait_send()
        for rdma in p2:
            rdma.wait_send()

    return pl.pallas_call(
        body,
        out_shape=jax.ShapeDtypeStruct((m, n), jnp.float32),
        in_specs=[
            pl.BlockSpec(memory_space=pltpu.VMEM),
            pl.BlockSpec(memory_space=pltpu.VMEM),
        ],
        out_specs=pl.BlockSpec(memory_space=pltpu.VMEM),
        scratch_shapes=[
            pltpu.VMEM((m, n), jnp.bfloat16),
            pltpu.VMEM((q, n), jnp.bfloat16),
            pltpu.VMEM((N_DEV - 1, q, n), jnp.bfloat16),
            pltpu.VMEM((N_DEV - 1, q, n), jnp.bfloat16),
            pltpu.SemaphoreType.DMA((N_DEV - 1,)),
            pltpu.SemaphoreType.DMA((N_DEV - 1,)),
            pltpu.SemaphoreType.DMA((N_DEV - 1,)),
            pltpu.SemaphoreType.DMA((N_DEV - 1,)),
        ],
        compiler_params=pltpu.CompilerParams(collective_id=0),
    )(A, B)


# device time: 24441 ns/iter; 3.6703x vs baseline; 1.0164x over previous
import jax
import jax.numpy as jnp
from jax import lax
from jax.experimental import pallas as pl
from jax.experimental.pallas import tpu as pltpu

N_DEV = 4


def kernel(A, B):
    m, k = A.shape
    k2, n = B.shape
    q = m // N_DEV

    def body(a_ref, b_ref, out_ref, stage_ref, comm1_ref,
             send_sems1, recv_sems1, send_sems2, recv_sems2):
        my_pos = lax.axis_index("i")

        barrier_sem = pltpu.get_barrier_semaphore()
        for off in range(1, N_DEV):
            pl.semaphore_signal(
                barrier_sem, inc=1,
                device_id=((my_pos + off) % N_DEV,),
                device_id_type=pl.DeviceIdType.MESH,
            )
        pl.semaphore_wait(barrier_sem, N_DEV - 1)

        partial = jnp.dot(
            a_ref[:, :].astype(jnp.bfloat16),
            b_ref[:, :].astype(jnp.bfloat16),
            preferred_element_type=jnp.float32,
        )
        stage_ref[:, :] = partial.astype(jnp.bfloat16)

        p1 = []
        for off in range(1, N_DEV):
            d = (my_pos + off) % N_DEV
            rdma = pltpu.make_async_remote_copy(
                src_ref=stage_ref.at[pl.ds(d * q, q), :],
                dst_ref=comm1_ref.at[off - 1],
                send_sem=send_sems1.at[off - 1],
                recv_sem=recv_sems1.at[off - 1],
                device_id=(d,),
                device_id_type=pl.DeviceIdType.MESH,
            )
            rdma.start()
            p1.append(rdma)

        for rdma in p1:
            rdma.wait_recv()
        acc = stage_ref[pl.ds(my_pos * q, q), :].astype(jnp.float32)
        for off in range(1, N_DEV):
            acc = acc + comm1_ref[off - 1, :, :].astype(jnp.float32)
        out_ref[pl.ds(my_pos * q, q), :] = acc.astype(jnp.bfloat16)

        p2 = []
        for off in range(1, N_DEV):
            d = (my_pos + off) % N_DEV
            rdma = pltpu.make_async_remote_copy(
                src_ref=out_ref.at[pl.ds(my_pos * q, q), :],
                dst_ref=out_ref.at[pl.ds(my_pos * q, q), :],
                send_sem=send_sems2.at[off - 1],
                recv_sem=recv_sems2.at[off - 1],
                device_id=(d,),
                device_id_type=pl.DeviceIdType.MESH,
            )
            rdma.start()
            p2.append(rdma)

        for off in range(1, N_DEV):
            src = (my_pos - off) % N_DEV
            recv = pltpu.make_async_remote_copy(
                src_ref=out_ref.at[pl.ds(src * q, q), :],
                dst_ref=out_ref.at[pl.ds(src * q, q), :],
                send_sem=send_sems2.at[off - 1],
                recv_sem=recv_sems2.at[off - 1],
                device_id=(src,),
                device_id_type=pl.DeviceIdType.MESH,
            )
            recv.wait_recv()

        for rdma in p1:
            rdma.wait_send()
        for rdma in p2:
            rdma.wait_send()

    return pl.pallas_call(
        body,
        out_shape=jax.ShapeDtypeStruct((m, n), jnp.bfloat16),
        in_specs=[
            pl.BlockSpec(memory_space=pltpu.VMEM),
            pl.BlockSpec(memory_space=pltpu.VMEM),
        ],
        out_specs=pl.BlockSpec(memory_space=pltpu.VMEM),
        scratch_shapes=[
            pltpu.VMEM((m, n), jnp.bfloat16),
            pltpu.VMEM((N_DEV - 1, q, n), jnp.bfloat16),
            pltpu.SemaphoreType.DMA((N_DEV - 1,)),
            pltpu.SemaphoreType.DMA((N_DEV - 1,)),
            pltpu.SemaphoreType.DMA((N_DEV - 1,)),
            pltpu.SemaphoreType.DMA((N_DEV - 1,)),
        ],
        compiler_params=pltpu.CompilerParams(collective_id=0),
    )(A, B)


# device time: 21376 ns/iter; 4.1966x vs baseline; 1.1434x over previous
import jax
import jax.numpy as jnp
from jax import lax
from jax.experimental import pallas as pl
from jax.experimental.pallas import tpu as pltpu

N_DEV = 4
C = 2


def kernel(A, B):
    m, k = A.shape
    k2, n = B.shape
    q = m // N_DEV
    nc = n // C

    def body(a_ref, b_ref, out_ref, stage_ref, comm1_ref,
             send_sems1, recv_sems1, send_sems2, recv_sems2):
        my_pos = lax.axis_index("i")

        barrier_sem = pltpu.get_barrier_semaphore()
        for off in range(1, N_DEV):
            pl.semaphore_signal(
                barrier_sem, inc=1,
                device_id=((my_pos + off) % N_DEV,),
                device_id_type=pl.DeviceIdType.MESH,
            )

        a_bf = a_ref[:, :].astype(jnp.bfloat16)

        def compute_chunk(c):
            partial = jnp.dot(
                a_bf,
                b_ref[:, pl.ds(c * nc, nc)].astype(jnp.bfloat16),
                preferred_element_type=jnp.float32,
            )
            stage_ref[:, pl.ds(c * nc, nc)] = partial.astype(jnp.bfloat16)

        def start_p1(c):
            rdmas = []
            for off in (2, 1, 3):
                d = (my_pos + off) % N_DEV
                rdma = pltpu.make_async_remote_copy(
                    src_ref=stage_ref.at[pl.ds(d * q, q), pl.ds(c * nc, nc)],
                    dst_ref=comm1_ref.at[off - 1, :, pl.ds(c * nc, nc)],
                    send_sem=send_sems1.at[off - 1, c],
                    recv_sem=recv_sems1.at[off - 1, c],
                    device_id=(d,),
                    device_id_type=pl.DeviceIdType.MESH,
                )
                rdma.start()
                rdmas.append(rdma)
            return rdmas

        def reduce_and_p2(c, p1_rdmas):
            for rdma in p1_rdmas:
                rdma.wait_recv()
            acc = stage_ref[pl.ds(my_pos * q, q), pl.ds(c * nc, nc)].astype(
                jnp.float32
            )
            for off in range(1, N_DEV):
                acc = acc + comm1_ref[
                    off - 1, :, pl.ds(c * nc, nc)
                ].astype(jnp.float32)
            out_ref[pl.ds(my_pos * q, q), pl.ds(c * nc, nc)] = acc.astype(
                jnp.bfloat16
            )
            rdmas = []
            for off in (2, 1, 3):
                d = (my_pos + off) % N_DEV
                rdma = pltpu.make_async_remote_copy(
                    src_ref=out_ref.at[pl.ds(my_pos * q, q), pl.ds(c * nc, nc)],
                    dst_ref=out_ref.at[pl.ds(my_pos * q, q), pl.ds(c * nc, nc)],
                    send_sem=send_sems2.at[off - 1, c],
                    recv_sem=recv_sems2.at[off - 1, c],
                    device_id=(d,),
                    device_id_type=pl.DeviceIdType.MESH,
                )
                rdma.start()
                rdmas.append(rdma)
            return rdmas

        compute_chunk(0)
        pl.semaphore_wait(barrier_sem, N_DEV - 1)
        p1 = [start_p1(0)]
        compute_chunk(1)
        p1.append(start_p1(1))
        p2 = [reduce_and_p2(0, p1[0])]
        p2.append(reduce_and_p2(1, p1[1]))

        for c in range(C):
            for off in range(1, N_DEV):
                src = (my_pos - off) % N_DEV
                recv = pltpu.make_async_remote_copy(
                    src_ref=out_ref.at[pl.ds(src * q, q), pl.ds(c * nc, nc)],
                    dst_ref=out_ref.at[pl.ds(src * q, q), pl.ds(c * nc, nc)],
                    send_sem=send_sems2.at[off - 1, c],
                    recv_sem=recv_sems2.at[off - 1, c],
                    device_id=(src,),
                    device_id_type=pl.DeviceIdType.MESH,
                )
                recv.wait_recv()

        for group in p1 + p2:
            for rdma in group:
                rdma.wait_send()

    return pl.pallas_call(
        body,
        out_shape=jax.ShapeDtypeStruct((m, n), jnp.bfloat16),
        in_specs=[
            pl.BlockSpec(memory_space=pltpu.VMEM),
            pl.BlockSpec(memory_space=pltpu.VMEM),
        ],
        out_specs=pl.BlockSpec(memory_space=pltpu.VMEM),
        scratch_shapes=[
            pltpu.VMEM((m, n), jnp.bfloat16),
            pltpu.VMEM((N_DEV - 1, q, n), jnp.bfloat16),
            pltpu.SemaphoreType.DMA((N_DEV - 1, C)),
            pltpu.SemaphoreType.DMA((N_DEV - 1, C)),
            pltpu.SemaphoreType.DMA((N_DEV - 1, C)),
            pltpu.SemaphoreType.DMA((N_DEV - 1, C)),
        ],
        compiler_params=pltpu.CompilerParams(collective_id=0),
    )(A, B)
